# baseline (speedup 1.0000x reference)
import functools

import jax
import jax.numpy as jnp
from jax import lax
from jax.experimental import pallas as pl
from jax.experimental.pallas import tpu as pltpu
from jax.experimental.pallas import tpu_sc as plsc

_C = 1000
_CP = 1024
_B = 16384
_D = 512
_NC = 2
_NS = 16
_L = 16
_W = _D // (_NC * _NS)
_H = _D // _NC
_SC = 1024
_NSC = _B // _SC
_SPT = _SC // _NS
_FB = 125
_NFB = _C // _FB


def _sc_stats(x, labs16, rm, rv, cc16):
    mesh = plsc.VectorSubcoreMesh(core_axis_name="c", subcore_axis_name="s")

    @functools.partial(
        pl.kernel,
        out_type=(
            jax.ShapeDtypeStruct((_C, _D), jnp.float32),
            jax.ShapeDtypeStruct((_C, _D), jnp.float32),
            jax.ShapeDtypeStruct((_CP, _L), jnp.float32),
        ),
        mesh=mesh,
        compiler_params=pltpu.CompilerParams(
            use_tc_tiling_on_sc=False, needs_layout_passes=False
        ),
        scratch_types=[
            pltpu.VMEM_SHARED((_SC, _H), jnp.float32),
            pltpu.VMEM_SHARED((_SC, _H), jnp.float32),
            pltpu.VMEM((_CP * _L,), jnp.float32),
            pltpu.VMEM((_CP * _L,), jnp.float32),
            pltpu.VMEM((_CP * _L,), jnp.float32),
            pltpu.VMEM((_B,), jnp.int32),
            pltpu.VMEM((256, _W), jnp.float32),
            pltpu.SemaphoreType.DMA,
            pltpu.SemaphoreType.DMA,
            pltpu.SemaphoreType.DMA,
            pltpu.VMEM((_FB, _L), jnp.float32),
            pltpu.VMEM((_FB, _L), jnp.float32),
            pltpu.VMEM((_FB, _L), jnp.float32),
            pltpu.VMEM((_FB, _L), jnp.float32),
            pltpu.VMEM((_FB, _L), jnp.float32),
            pltpu.VMEM((_FB, _L), jnp.float32),
        ],
    )
    def k(x_h, lab_h, rm_h, rv_h, cc_h, om_h, ov_h, oc_h,
          stg0, stg1, sum_t, ss_t, cnt_t, lv, xv, sg0, sg1, slb,
          rm_b, rv_b, cc_b, om_b, ov_b, on_b):
        cid = lax.axis_index("c")
        sid = lax.axis_index("s")
        hb = cid * _H
        cb = hb + sid * _W

        def stage_cp(sc, stg, sem):
            return pltpu.make_async_copy(
                x_h.at[pl.ds(sc * _SC + sid * _SPT, _SPT), pl.ds(hb, _H)],
                stg.at[pl.ds(sid * _SPT, _SPT)],
                sem,
            )

        lab_cp = pltpu.make_async_copy(lab_h, lv, slb)

        lab_cp.start()

        zero = jnp.zeros((_L,), jnp.float32)

        @plsc.parallel_loop(0, _CP, unroll=8)
        def _(i):
            sl = pl.ds(i * _L, _L)
            sum_t[sl] = zero
            ss_t[sl] = zero
            cnt_t[sl] = zero

        lab_cp.wait()

        one = jnp.ones((_L,), jnp.float32)
        lanes = lax.iota(jnp.int32, _L)
        consts_r = [jnp.full((_L,), r, jnp.int32) for r in range(_L)]

        def consume(sc):
            @plsc.parallel_loop(0, 256 // _L, unroll=2)
            def _(g):
                g0 = sc * 256 + g * _L
                l16 = lv[pl.ds(g0, _L)]
                plsc.addupdate_scatter(cnt_t, [l16], one)
                gbase = jnp.full((_L,), g0, jnp.int32)
                for r in range(_L):
                    i = g * _L + r
                    a = plsc.load_gather(lv, [gbase + consts_r[r]])
                    addr = a + lanes
                    v = xv[i, pl.ds(0, _W)]
                    plsc.addupdate_scatter(sum_t, [addr], v)
                    plsc.addupdate_scatter(ss_t, [addr], v * v)

        def super_chunk(sc, _):
            pltpu.sync_copy(
                x_h.at[pl.ds(sc * 256, 256), pl.ds(cb, _W)], xv
            )
            consume(sc)
            return 0

        lax.fori_loop(0, _B // 256, super_chunk, 0)

        def blk(b, _):
            r0 = b * _FB
            pltpu.sync_copy(rm_h.at[pl.ds(r0, _FB), pl.ds(cb, _W)], rm_b)
            pltpu.sync_copy(rv_h.at[pl.ds(r0, _FB), pl.ds(cb, _W)], rv_b)
            pltpu.sync_copy(cc_h.at[pl.ds(r0, _FB)], cc_b)

            @plsc.parallel_loop(0, _FB, unroll=5)
            def _(i):
                r = r0 + i
                sl = pl.ds(r * _L, _L)
                nb = plsc.load_gather(cnt_t, [jnp.full((_L,), r, jnp.int32) * _L])
                na = cc_b[i, pl.ds(0, _L)]
                n = na + nb
                on_b[i, pl.ds(0, _L)] = n
                rn = 1.0 / jnp.maximum(n, 1.0)
                s_ = sum_t[sl]
                q_ = ss_t[sl]
                m_ = rm_b[i, pl.ds(0, _L)]
                v_ = rv_b[i, pl.ds(0, _L)]
                mean = (na * m_ + s_) * rn
                om_b[i, pl.ds(0, _L)] = mean
                ov_b[i, pl.ds(0, _L)] = (na * (v_ + m_ * m_) + q_) * rn - mean * mean

            pltpu.sync_copy(om_b, om_h.at[pl.ds(r0, _FB), pl.ds(cb, _W)])
            pltpu.sync_copy(ov_b, ov_h.at[pl.ds(r0, _FB), pl.ds(cb, _W)])

            @pl.when(jnp.logical_and(cid == 0, sid == 0))
            def _():
                pltpu.sync_copy(on_b, oc_h.at[pl.ds(r0, _FB)])

            return 0

        lax.fori_loop(0, _NFB, blk, 0)

    return k(x, labs16, rm, rv, cc16)


def kernel(x, labels, running_mean, running_var, class_count):
    cc16 = jnp.pad(
        jnp.broadcast_to(class_count, (_C, _L)), ((0, _CP - _C), (0, 0))
    )
    labs16 = labels.astype(jnp.int32) * _L
    um, uv, cn = _sc_stats(x, labs16, running_mean, running_var, cc16)
    return um, uv, cn[:_C, :1]
